# baseline (speedup 1.0000x reference)
import functools

import jax
import jax.numpy as jnp
from jax import lax
from jax.experimental import pallas as pl
from jax.experimental.pallas import tpu as pltpu
from jax.experimental.pallas import tpu_sc as plsc

CHUNK = 16
NSTATES = 65536
NC, NS, L = 2, 16, 16
NW = NC * NS
BR, BC = 8, 1024
BLK = BR * BC
NBUF = 2


def _pack_indices(states):
    nrow, ncol = states.shape
    s = states.astype(jnp.int16)
    s = s.reshape(2, nrow // 16, BR, ncol // BC, BC)
    s = s.transpose(0, 1, 3, 2, 4)
    s = s.reshape(-1, 2, L).transpose(0, 2, 1)
    return s.reshape(-1)


def kernel(states, lut):
    nrow, ncol = states.shape
    blk_per_tr = ncol // BC
    tr_per_w = nrow // 8 // 2
    nblk = tr_per_w * blk_per_tr
    half_elems = nblk * BLK
    lut_flat = lut.reshape(-1)
    states_p = _pack_indices(states)

    mesh = plsc.VectorSubcoreMesh(core_axis_name="c", subcore_axis_name="s")

    @functools.partial(
        pl.kernel,
        out_type=jax.ShapeDtypeStruct((CHUNK, nrow, ncol), jnp.float32),
        mesh=mesh,
        scratch_types=[
            pltpu.VMEM((NSTATES,), jnp.float32),
            [pltpu.VMEM((BLK,), jnp.int16)] * NBUF,
            pltpu.VMEM((NBUF, BR, BC), jnp.float32),
            pltpu.SemaphoreType.DMA,
            [pltpu.SemaphoreType.DMA] * NBUF,
            [pltpu.SemaphoreType.DMA] * NBUF,
        ],
        compiler_params=pltpu.CompilerParams(
            needs_layout_passes=False, use_tc_tiling_on_sc=True),
    )
    def k(states_hbm, lut_hbm, out_hbm, lut_v, idx_v, res_v, lut_sem,
          in_sems, out_sems):
        wid = lax.axis_index("s") * NC + lax.axis_index("c")
        row = wid // 2
        half = wid % 2

        lut_cp = pltpu.async_copy(
            lut_hbm.at[pl.ds(row * NSTATES, NSTATES)], lut_v, lut_sem)

        def in_cp(b, j):
            return pltpu.make_async_copy(
                states_hbm.at[pl.ds(half * half_elems + b * BLK, BLK)],
                idx_v[j], in_sems[j])

        def out_cp(b, j):
            tr = half * tr_per_w + b // blk_per_tr
            c0 = (b % blk_per_tr) * BC
            return pltpu.make_async_copy(
                res_v.at[j],
                out_hbm.at[row, pl.ds(tr * BR, BR), pl.ds(c0, BC)],
                out_sems[j])

        for j in range(NBUF):
            in_cp(j, j).start()
        lut_cp.wait()

        def blk_body(i, carry):
            for j in range(NBUF):
                b = i * NBUF + j
                in_cp(b, j).wait()
                pl.when(b >= NBUF)(lambda: out_cp(b - NBUF, j).wait())

                @plsc.parallel_loop(0, BC, step=2 * L, unroll=2)
                def g_body(g):
                    for r in range(BR):
                        ab = idx_v[j][pl.ds(r * BC + g, 2 * L)]
                        lo, hi = plsc.unpack(
                            ab, format=plsc.PackFormat.INTERLEAVED)
                        res_v[j, r, pl.ds(g, L)] = plsc.load_gather(
                            lut_v, [lo & 0xFFFF])
                        res_v[j, r, pl.ds(g + L, L)] = plsc.load_gather(
                            lut_v, [hi & 0xFFFF])

                out_cp(b, j).start()
                pl.when(b + NBUF < nblk)(lambda: in_cp(b + NBUF, j).start())
            return carry

        lax.fori_loop(0, nblk // NBUF, blk_body, 0)
        for j in range(NBUF):
            out_cp(nblk - NBUF + j, j).wait()

    return k(states_p, lut_flat)


# speedup vs baseline: 3.9104x; 3.9104x over previous
import functools

import jax
import jax.numpy as jnp
from jax import lax
from jax.experimental import pallas as pl
from jax.experimental.pallas import tpu as pltpu
from jax.experimental.pallas import tpu_sc as plsc

CHUNK = 16
NSTATES = 65536
NC, NS, L = 2, 16, 16
NW = NC * NS
BR, BC = 8, 1024
PC = BC // 2
NBUF = 2


def _pack_indices(states):
    nrow, ncol = states.shape
    v = states.reshape(nrow, ncol // (2 * L), 2, L)
    packed = v[:, :, 0, :] | (v[:, :, 1, :] << 16)
    return packed.reshape(nrow, ncol // 2)


def kernel(states, lut):
    nrow, ncol = states.shape
    blk_per_tr = ncol // BC
    tr_per_w = nrow // 8 // 2
    nblk = tr_per_w * blk_per_tr
    lut_flat = lut.reshape(-1)
    states_p = _pack_indices(states)

    mesh = plsc.VectorSubcoreMesh(core_axis_name="c", subcore_axis_name="s")

    @functools.partial(
        pl.kernel,
        out_type=jax.ShapeDtypeStruct((CHUNK, nrow, ncol), jnp.float32),
        mesh=mesh,
        scratch_types=[
            pltpu.VMEM((NSTATES,), jnp.float32),
            pltpu.VMEM((NBUF, BR, PC), jnp.int32),
            pltpu.VMEM((NBUF, BR, BC), jnp.float32),
            pltpu.SemaphoreType.DMA,
            [pltpu.SemaphoreType.DMA] * NBUF,
            [pltpu.SemaphoreType.DMA] * NBUF,
        ],
        compiler_params=pltpu.CompilerParams(
            needs_layout_passes=False, use_tc_tiling_on_sc=True),
    )
    def k(states_hbm, lut_hbm, out_hbm, lut_v, idx_v, res_v, lut_sem,
          in_sems, out_sems):
        wid = lax.axis_index("s") * NC + lax.axis_index("c")
        row = wid // 2
        half = wid % 2

        lut_cp = pltpu.async_copy(
            lut_hbm.at[pl.ds(row * NSTATES, NSTATES)], lut_v, lut_sem)

        def blk_pos(b):
            tr = half * tr_per_w + b // blk_per_tr
            c0 = (b % blk_per_tr)
            return tr, c0

        def in_cp(b, j):
            tr, c0 = blk_pos(b)
            return pltpu.make_async_copy(
                states_hbm.at[pl.ds(tr * BR, BR), pl.ds(c0 * PC, PC)],
                idx_v.at[j], in_sems[j])

        def out_cp(b, j):
            tr, c0 = blk_pos(b)
            return pltpu.make_async_copy(
                res_v.at[j],
                out_hbm.at[row, pl.ds(tr * BR, BR), pl.ds(c0 * BC, BC)],
                out_sems[j])

        for j in range(NBUF):
            in_cp(j, j).start()
        lut_cp.wait()

        def blk_body(i, carry):
            for j in range(NBUF):
                b = i * NBUF + j
                in_cp(b, j).wait()
                pl.when(b >= NBUF)(lambda: out_cp(b - NBUF, j).wait())

                @plsc.parallel_loop(0, PC, step=L, unroll=2)
                def g_body(g):
                    for r in range(BR):
                        w = idx_v[j, r, pl.ds(g, L)]
                        hi = plsc.bitcast(
                            plsc.bitcast(w, jnp.uint32) >> 16, jnp.int32)
                        res_v[j, r, pl.ds(2 * g, L)] = plsc.load_gather(
                            lut_v, [w & 0xFFFF])
                        res_v[j, r, pl.ds(2 * g + L, L)] = plsc.load_gather(
                            lut_v, [hi])

                out_cp(b, j).start()
                pl.when(b + NBUF < nblk)(lambda: in_cp(b + NBUF, j).start())
            return carry

        lax.fori_loop(0, nblk // NBUF, blk_body, 0)
        for j in range(NBUF):
            out_cp(nblk - NBUF + j, j).wait()

    return k(states_p, lut_flat)


# speedup vs baseline: 4.0389x; 1.0329x over previous
import functools

import jax
import jax.numpy as jnp
from jax import lax
from jax.experimental import pallas as pl
from jax.experimental.pallas import tpu as pltpu
from jax.experimental.pallas import tpu_sc as plsc

CHUNK = 16
NSTATES = 65536
NC, NS, L = 2, 16, 16
NW = NC * NS
BR, BC = 8, 1024
NBUF = 2


def kernel(states, lut):
    nrow, ncol = states.shape
    blk_per_tr = ncol // BC
    tr_per_w = nrow // 8 // 2
    nblk = tr_per_w * blk_per_tr
    lut_flat = lut.reshape(-1)

    mesh = plsc.VectorSubcoreMesh(core_axis_name="c", subcore_axis_name="s")

    @functools.partial(
        pl.kernel,
        out_type=jax.ShapeDtypeStruct((CHUNK, nrow, ncol), jnp.float32),
        mesh=mesh,
        scratch_types=[
            pltpu.VMEM((NSTATES,), jnp.float32),
            pltpu.VMEM((NBUF, BR, BC), jnp.int32),
            pltpu.VMEM((NBUF, BR, BC), jnp.float32),
            pltpu.SemaphoreType.DMA,
            [pltpu.SemaphoreType.DMA] * NBUF,
            [pltpu.SemaphoreType.DMA] * NBUF,
        ],
        compiler_params=pltpu.CompilerParams(
            needs_layout_passes=False, use_tc_tiling_on_sc=True),
    )
    def k(states_hbm, lut_hbm, out_hbm, lut_v, idx_v, res_v, lut_sem,
          in_sems, out_sems):
        wid = lax.axis_index("s") * NC + lax.axis_index("c")
        row = wid // 2
        half = wid % 2

        lut_cp = pltpu.async_copy(
            lut_hbm.at[pl.ds(row * NSTATES, NSTATES)], lut_v, lut_sem)

        def blk_slc(b):
            tr = half * tr_per_w + b // blk_per_tr
            c0 = (b % blk_per_tr) * BC
            return pl.ds(tr * BR, BR), pl.ds(c0, BC)

        def in_cp(b, j):
            r, c = blk_slc(b)
            return pltpu.make_async_copy(
                states_hbm.at[r, c], idx_v.at[j], in_sems[j])

        def out_cp(b, j):
            r, c = blk_slc(b)
            return pltpu.make_async_copy(
                res_v.at[j], out_hbm.at[row, r, c], out_sems[j])

        for j in range(NBUF):
            in_cp(j, j).start()
        lut_cp.wait()

        def blk_body(i, carry):
            for j in range(NBUF):
                b = i * NBUF + j
                in_cp(b, j).wait()
                pl.when(b >= NBUF)(lambda: out_cp(b - NBUF, j).wait())

                @plsc.parallel_loop(0, BC, step=L, unroll=2)
                def g_body(g):
                    for r in range(BR):
                        iv = idx_v[j, r, pl.ds(g, L)]
                        res_v[j, r, pl.ds(g, L)] = plsc.load_gather(
                            lut_v, [iv])

                out_cp(b, j).start()
                pl.when(b + NBUF < nblk)(lambda: in_cp(b + NBUF, j).start())
            return carry

        lax.fori_loop(0, nblk // NBUF, blk_body, 0)
        for j in range(NBUF):
            out_cp(nblk - NBUF + j, j).wait()

    return k(states, lut_flat)
